# baseline (speedup 1.0000x reference)
import functools

import jax
import jax.numpy as jnp
from jax import lax
from jax.experimental import pallas as pl
from jax.experimental.pallas import tpu as pltpu
from jax.experimental.pallas import tpu_sc as plsc

N = 10000
E = 320000
D = 128
H = 128

NC = 2
NS = 16
NW = NC * NS
CHUNK = 128
NCH_T = 80
EPAD = NW * NCH_T * CHUNK
NP_ = 10240
ROWS_PER_SUB = NP_ // NS

_MM = functools.partial(jnp.dot, precision=lax.Precision.HIGHEST,
                        preferred_element_type=jnp.float32)

_MESH = dict(core_axis_name="c", subcore_axis_name="s")


@functools.lru_cache(maxsize=None)
def _make_sc_agg():
    mesh = plsc.VectorSubcoreMesh(num_cores=NC, num_subcores=NS, **_MESH)
    scratch = [
        pltpu.VMEM((CHUNK,), jnp.int32),
        pltpu.VMEM((CHUNK,), jnp.int32),
        pltpu.VMEM((CHUNK,), jnp.int32),
        pltpu.VMEM((CHUNK,), jnp.int32),
        pltpu.VMEM((CHUNK, D), jnp.float32),
        pltpu.VMEM((CHUNK, D), jnp.float32),
        pltpu.VMEM_SHARED((NP_, D), jnp.float32),
        pltpu.SemaphoreType.DMA,
        pltpu.SemaphoreType.DMA,
    ]

    @functools.partial(
        pl.kernel, mesh=mesh, scratch_types=scratch,
        out_type=jax.ShapeDtypeStruct((NC, NP_, D), jnp.float32))
    def sc_agg(h_hbm, src1_hbm, dst1_hbm, zd_hbm, agg_out,
               src_a, src_b, dst_a, dst_b, rows_a, rows_b,
               acc_sh, sem_a, sem_b):
        cid = lax.axis_index("c")
        sid = lax.axis_index("s")
        wid = sid * NC + cid
        row0 = sid * ROWS_PER_SUB

        pltpu.sync_copy(zd_hbm, rows_a)

        @pl.loop(0, ROWS_PER_SUB // CHUNK)
        def _(i):
            pltpu.sync_copy(rows_a, acc_sh.at[pl.ds(row0 + i * CHUNK, CHUNK)])

        plsc.subcore_barrier()

        def load_idx(sbuf, dbuf, i):
            off = (wid + i * NW) * CHUNK
            pltpu.sync_copy(src1_hbm.at[pl.ds(off, CHUNK)], sbuf)
            pltpu.sync_copy(dst1_hbm.at[pl.ds(off, CHUNK)], dbuf)

        def start_gather(idx_buf, rows_buf, sem):
            pltpu.async_copy(h_hbm.at[idx_buf], rows_buf, sem)

        def finish_gather(idx_buf, rows_buf, sem):
            pltpu.make_async_copy(h_hbm.at[idx_buf], rows_buf, sem).wait()

        def scatter(rows_buf, dbuf):
            pltpu.sync_copy(rows_buf, acc_sh.at[dbuf], add=True)

        load_idx(src_a, dst_a, 0)
        start_gather(src_a, rows_a, sem_a)

        @pl.loop(0, NCH_T // 2 - 1)
        def _(p):
            load_idx(src_b, dst_b, 2 * p + 1)
            start_gather(src_b, rows_b, sem_b)
            finish_gather(src_a, rows_a, sem_a)
            scatter(rows_a, dst_a)
            load_idx(src_a, dst_a, 2 * p + 2)
            start_gather(src_a, rows_a, sem_a)
            finish_gather(src_b, rows_b, sem_b)
            scatter(rows_b, dst_b)

        load_idx(src_b, dst_b, NCH_T - 1)
        start_gather(src_b, rows_b, sem_b)
        finish_gather(src_a, rows_a, sem_a)
        scatter(rows_a, dst_a)
        finish_gather(src_b, rows_b, sem_b)
        scatter(rows_b, dst_b)

        plsc.subcore_barrier()
        rows = pl.ds(row0, ROWS_PER_SUB)
        pltpu.sync_copy(acc_sh.at[rows], agg_out.at[cid, rows])

    return sc_agg


@functools.lru_cache(maxsize=None)
def _make_sc_cnt():
    mesh = plsc.VectorSubcoreMesh(num_cores=NC, num_subcores=NS, **_MESH)
    scratch = [
        pltpu.VMEM((NCH_T, CHUNK), jnp.int32),
        pltpu.VMEM((CHUNK, D), jnp.float32),
        pltpu.VMEM((CHUNK, D), jnp.float32),
        pltpu.VMEM_SHARED((NP_, D), jnp.float32),
        pltpu.SemaphoreType.DMA,
    ]

    @functools.partial(
        pl.kernel, mesh=mesh, scratch_types=scratch,
        out_type=jax.ShapeDtypeStruct((NC, NP_, D), jnp.float32))
    def sc_cnt(dst2_hbm, zd_hbm, ones_hbm, cnt_out,
               dst2v, ones_v, zbuf, cnt_sh, sem):
        cid = lax.axis_index("c")
        sid = lax.axis_index("s")
        wid = sid * NC + cid
        row0 = sid * ROWS_PER_SUB
        cbase = wid * NCH_T

        pltpu.sync_copy(zd_hbm, zbuf)
        pltpu.sync_copy(ones_hbm, ones_v)
        pltpu.sync_copy(dst2_hbm.at[pl.ds(cbase, NCH_T)], dst2v)

        @pl.loop(0, ROWS_PER_SUB // CHUNK)
        def _(i):
            pltpu.sync_copy(zbuf, cnt_sh.at[pl.ds(row0 + i * CHUNK, CHUNK)])

        plsc.subcore_barrier()

        @pl.loop(0, NCH_T // 16)
        def _(w):
            @pl.loop(0, 16)
            def _(i):
                pltpu.async_copy(ones_v, cnt_sh.at[dst2v.at[w * 16 + i]],
                                 sem, add=True)

            @pl.loop(0, 16)
            def _(i):
                pltpu.make_async_copy(ones_v, cnt_sh.at[dst2v.at[w * 16 + i]],
                                      sem).wait()

        plsc.subcore_barrier()
        rows = pl.ds(row0, ROWS_PER_SUB)
        pltpu.sync_copy(cnt_sh.at[rows], cnt_out.at[cid, rows])

    return sc_cnt


def _sc_agg(h, src2, dst2, zd):
    return _make_sc_agg()(h, src2, dst2, zd)


def _sc_cnt(dst2, zd, ones):
    return _make_sc_cnt()(dst2, zd, ones)


ROWS = 1000
_GRID = N // ROWS


def _rows_spec(minor):
    return pl.BlockSpec((ROWS, minor), lambda i: (i, 0))


def _full_spec(shape):
    nd = len(shape)
    return pl.BlockSpec(shape, lambda i, _nd=nd: (0,) * _nd)


def _norm_res_body(x_ref, wp_ref, bp_ref, xn_ref, res_ref):
    x = x_ref[...]
    nrm = jnp.sqrt(jnp.sum(x * x, axis=1, keepdims=True))
    xn = x / jnp.maximum(nrm, 1e-12)
    xn_ref[...] = xn
    res_ref[...] = _MM(xn, wp_ref[...]) + bp_ref[...]


_norm_res = pl.pallas_call(
    _norm_res_body,
    grid=(_GRID,),
    in_specs=[_rows_spec(D), _full_spec((D, H)), _full_spec((1, H))],
    out_specs=[_rows_spec(D), _rows_spec(H)],
    out_shape=[jax.ShapeDtypeStruct((N, D), jnp.float32),
               jax.ShapeDtypeStruct((N, H), jnp.float32)],
)


def _layer_body(a_ref, c_ref, h_ref, res_ref, wl_ref, bl_ref, wr_ref, out_ref):
    agg = a_ref[0] + a_ref[1]
    cnt = c_ref[0, :, 0:1] + c_ref[1, :, 0:1]
    mean = agg / jnp.maximum(cnt, 1.0)
    h = h_ref[...]
    pre = _MM(mean, wl_ref[...]) + bl_ref[...] + _MM(h, wr_ref[...])
    out_ref[...] = jnp.maximum(pre, 0.0) + res_ref[...]


def _agg_spec():
    return pl.BlockSpec((NC, ROWS, D), lambda i: (0, i, 0))


_sage_layer = pl.pallas_call(
    _layer_body,
    grid=(_GRID,),
    in_specs=[_agg_spec(), _agg_spec(), _rows_spec(D), _rows_spec(H),
              _full_spec((D, H)), _full_spec((1, H)), _full_spec((D, H))],
    out_specs=_rows_spec(H),
    out_shape=jax.ShapeDtypeStruct((N, H), jnp.float32),
)


def _layer2_head_body(a_ref, c_ref, h_ref, wl_ref, bl_ref, wr_ref,
                      rr_ref, ws1_ref, bs1_ref, ws2_ref, bs2_ref,
                      alpha_ref, out_ref):
    agg = a_ref[0] + a_ref[1]
    cnt = c_ref[0, :, 0:1] + c_ref[1, :, 0:1]
    mean = agg / jnp.maximum(cnt, 1.0)
    h = h_ref[...]
    pre = _MM(mean, wl_ref[...]) + bl_ref[...] + _MM(h, wr_ref[...])
    h2 = jnp.maximum(pre, 0.0) + h
    t = jnp.maximum(_MM(h2, ws1_ref[...]) + bs1_ref[...], 0.0)
    g = jnp.sum(t * ws2_ref[...], axis=1, keepdims=True) + bs2_ref[...]
    alpha = alpha_ref[...]
    out_ref[...] = alpha * rr_ref[...] + (1.0 - alpha) * g


_layer2_head = pl.pallas_call(
    _layer2_head_body,
    grid=(_GRID,),
    in_specs=[_agg_spec(), _agg_spec(), _rows_spec(H),
              _full_spec((D, H)), _full_spec((1, H)), _full_spec((D, H)),
              _rows_spec(1), _full_spec((H, H // 2)),
              _full_spec((1, H // 2)), _full_spec((1, H // 2)),
              _full_spec((1, 1)), _full_spec((1, 1))],
    out_specs=_rows_spec(1),
    out_shape=jax.ShapeDtypeStruct((N, 1), jnp.float32),
)


def kernel(x, edge_index, reranker_scores, Wp, bp, Wl0, bl0, Wr0, Wl1, bl1,
           Wr1, Ws1, bs1, Ws2, bs2, alpha_logit):
    zd = jnp.zeros((CHUNK, D), jnp.float32)
    ones = jnp.ones((CHUNK, D), jnp.float32)

    pad = EPAD - E
    psrc = jnp.arange(pad, dtype=jnp.int32) % N
    src1 = jnp.concatenate([edge_index[0], psrc])
    dump = N + (jnp.arange(pad, dtype=jnp.int32) % (NP_ - N))
    dst1 = jnp.concatenate([edge_index[1], dump])
    dst2 = dst1.reshape(-1, CHUNK)

    cnt = _sc_cnt(dst2, zd, ones)
    xn, res = _norm_res(x, Wp, bp.reshape(1, H))
    agg0 = _sc_agg(xn, src1, dst1, zd)
    h1 = _sage_layer(agg0, cnt, xn, res, Wl0, bl0.reshape(1, H), Wr0)
    agg1 = _sc_agg(h1, src1, dst1, zd)
    alpha = jax.nn.sigmoid(alpha_logit).reshape(1, 1)
    out = _layer2_head(agg1, cnt, h1, Wl1, bl1.reshape(1, H), Wr1,
                       reranker_scores.reshape(N, 1), Ws1,
                       bs1.reshape(1, H // 2), Ws2.reshape(1, H // 2),
                       bs2.reshape(1, 1), alpha)
    return out[:, 0]


# speedup vs baseline: 1.1174x; 1.1174x over previous
import functools

import numpy as np

import jax
import jax.numpy as jnp
from jax import lax
from jax.experimental import pallas as pl
from jax.experimental.pallas import tpu as pltpu
from jax.experimental.pallas import tpu_sc as plsc

N = 10000
E = 320000
D = 128
H = 128

NC = 2
NS = 16
NW = NC * NS
CHUNK = 128
NCH_T = 80
EPAD = NW * NCH_T * CHUNK
NP_ = 10240
ROWS_PER_SUB = NP_ // NS

_MM = functools.partial(jnp.dot, precision=lax.Precision.HIGHEST,
                        preferred_element_type=jnp.float32)

_MESH = dict(core_axis_name="c", subcore_axis_name="s")

_NCHUNKS = EPAD // CHUNK
_PERM = np.arange(_NCHUNKS).reshape(NCH_T, NW).T.reshape(-1)


@functools.lru_cache(maxsize=None)
def _make_sc_agg():
    mesh = plsc.VectorSubcoreMesh(num_cores=NC, num_subcores=NS, **_MESH)
    scratch = [
        pltpu.VMEM((NCH_T, CHUNK), jnp.int32),
        pltpu.VMEM((CHUNK,), jnp.int32),
        pltpu.VMEM((CHUNK,), jnp.int32),
        pltpu.VMEM((CHUNK, D), jnp.float32),
        pltpu.VMEM((CHUNK, D), jnp.float32),
        pltpu.VMEM_SHARED((NP_, D), jnp.float32),
        pltpu.SemaphoreType.DMA,
        pltpu.SemaphoreType.DMA,
    ]

    @functools.partial(
        pl.kernel, mesh=mesh, scratch_types=scratch,
        out_type=jax.ShapeDtypeStruct((NC, NP_, D), jnp.float32))
    def sc_agg(h_hbm, src1_hbm, dst2_hbm, zd_hbm, agg_out,
               dst2v, src_a, src_b, rows_a, rows_b, acc_sh, sem_a, sem_b):
        cid = lax.axis_index("c")
        sid = lax.axis_index("s")
        wid = sid * NC + cid
        row0 = sid * ROWS_PER_SUB
        cbase = wid * NCH_T

        pltpu.sync_copy(zd_hbm, rows_a)

        @pl.loop(0, ROWS_PER_SUB // CHUNK)
        def _(i):
            pltpu.sync_copy(rows_a, acc_sh.at[pl.ds(row0 + i * CHUNK, CHUNK)])

        pltpu.sync_copy(dst2_hbm.at[pl.ds(cbase, NCH_T)], dst2v)
        plsc.subcore_barrier()

        def load_src(buf, c):
            pltpu.sync_copy(src1_hbm.at[pl.ds(c * CHUNK, CHUNK)], buf)

        def start_gather(idx_buf, rows_buf, sem):
            pltpu.async_copy(h_hbm.at[idx_buf], rows_buf, sem)

        def finish_gather(idx_buf, rows_buf, sem):
            pltpu.make_async_copy(h_hbm.at[idx_buf], rows_buf, sem).wait()

        def scatter(rows_buf, c):
            pltpu.sync_copy(rows_buf, acc_sh.at[dst2v.at[c - cbase]], add=True)

        load_src(src_a, cbase)
        start_gather(src_a, rows_a, sem_a)

        @pl.loop(0, NCH_T // 2 - 1)
        def _(p):
            c0 = cbase + 2 * p
            load_src(src_b, c0 + 1)
            start_gather(src_b, rows_b, sem_b)
            finish_gather(src_a, rows_a, sem_a)
            scatter(rows_a, c0)
            load_src(src_a, c0 + 2)
            start_gather(src_a, rows_a, sem_a)
            finish_gather(src_b, rows_b, sem_b)
            scatter(rows_b, c0 + 1)

        clast = cbase + NCH_T - 1
        load_src(src_b, clast)
        start_gather(src_b, rows_b, sem_b)
        finish_gather(src_a, rows_a, sem_a)
        scatter(rows_a, clast - 1)
        finish_gather(src_b, rows_b, sem_b)
        scatter(rows_b, clast)

        plsc.subcore_barrier()
        rows = pl.ds(row0, ROWS_PER_SUB)
        pltpu.sync_copy(acc_sh.at[rows], agg_out.at[cid, rows])

    return sc_agg


@functools.lru_cache(maxsize=None)
def _make_sc_cnt():
    mesh = plsc.VectorSubcoreMesh(num_cores=NC, num_subcores=NS, **_MESH)
    scratch = [
        pltpu.VMEM((NCH_T, CHUNK), jnp.int32),
        pltpu.VMEM((CHUNK, D), jnp.float32),
        pltpu.VMEM((CHUNK, D), jnp.float32),
        pltpu.VMEM_SHARED((NP_, D), jnp.float32),
        pltpu.SemaphoreType.DMA,
    ]

    @functools.partial(
        pl.kernel, mesh=mesh, scratch_types=scratch,
        out_type=jax.ShapeDtypeStruct((NC, NP_, D), jnp.float32))
    def sc_cnt(dst2_hbm, zd_hbm, ones_hbm, cnt_out,
               dst2v, ones_v, zbuf, cnt_sh, sem):
        cid = lax.axis_index("c")
        sid = lax.axis_index("s")
        wid = sid * NC + cid
        row0 = sid * ROWS_PER_SUB
        cbase = wid * NCH_T

        pltpu.sync_copy(zd_hbm, zbuf)
        pltpu.sync_copy(ones_hbm, ones_v)
        pltpu.sync_copy(dst2_hbm.at[pl.ds(cbase, NCH_T)], dst2v)

        @pl.loop(0, ROWS_PER_SUB // CHUNK)
        def _(i):
            pltpu.sync_copy(zbuf, cnt_sh.at[pl.ds(row0 + i * CHUNK, CHUNK)])

        plsc.subcore_barrier()

        @pl.loop(0, NCH_T // 16)
        def _(w):
            @pl.loop(0, 16)
            def _(i):
                pltpu.async_copy(ones_v, cnt_sh.at[dst2v.at[w * 16 + i]],
                                 sem, add=True)

            @pl.loop(0, 16)
            def _(i):
                pltpu.make_async_copy(ones_v, cnt_sh.at[dst2v.at[w * 16 + i]],
                                      sem).wait()

        plsc.subcore_barrier()
        rows = pl.ds(row0, ROWS_PER_SUB)
        pltpu.sync_copy(cnt_sh.at[rows], cnt_out.at[cid, rows])

    return sc_cnt


def _sc_agg(h, src2, dst2, zd):
    return _make_sc_agg()(h, src2, dst2, zd)


def _sc_cnt(dst2, zd, ones):
    return _make_sc_cnt()(dst2, zd, ones)


ROWS = 1000
_GRID = N // ROWS


def _rows_spec(minor):
    return pl.BlockSpec((ROWS, minor), lambda i: (i, 0))


def _full_spec(shape):
    nd = len(shape)
    return pl.BlockSpec(shape, lambda i, _nd=nd: (0,) * _nd)


def _norm_res_body(x_ref, wp_ref, bp_ref, xn_ref, res_ref):
    x = x_ref[...]
    nrm = jnp.sqrt(jnp.sum(x * x, axis=1, keepdims=True))
    xn = x / jnp.maximum(nrm, 1e-12)
    xn_ref[...] = xn
    res_ref[...] = _MM(xn, wp_ref[...]) + bp_ref[...]


_norm_res = pl.pallas_call(
    _norm_res_body,
    grid=(_GRID,),
    in_specs=[_rows_spec(D), _full_spec((D, H)), _full_spec((1, H))],
    out_specs=[_rows_spec(D), _rows_spec(H)],
    out_shape=[jax.ShapeDtypeStruct((N, D), jnp.float32),
               jax.ShapeDtypeStruct((N, H), jnp.float32)],
)


def _layer_body(a_ref, c_ref, h_ref, res_ref, wl_ref, bl_ref, wr_ref, out_ref):
    agg = a_ref[0] + a_ref[1]
    cnt = c_ref[0, :, 0:1] + c_ref[1, :, 0:1]
    mean = agg / jnp.maximum(cnt, 1.0)
    h = h_ref[...]
    pre = _MM(mean, wl_ref[...]) + bl_ref[...] + _MM(h, wr_ref[...])
    out_ref[...] = jnp.maximum(pre, 0.0) + res_ref[...]


def _agg_spec():
    return pl.BlockSpec((NC, ROWS, D), lambda i: (0, i, 0))


_sage_layer = pl.pallas_call(
    _layer_body,
    grid=(_GRID,),
    in_specs=[_agg_spec(), _agg_spec(), _rows_spec(D), _rows_spec(H),
              _full_spec((D, H)), _full_spec((1, H)), _full_spec((D, H))],
    out_specs=_rows_spec(H),
    out_shape=jax.ShapeDtypeStruct((N, H), jnp.float32),
)


def _layer2_head_body(a_ref, c_ref, h_ref, wl_ref, bl_ref, wr_ref,
                      rr_ref, ws1_ref, bs1_ref, ws2_ref, bs2_ref,
                      alpha_ref, out_ref):
    agg = a_ref[0] + a_ref[1]
    cnt = c_ref[0, :, 0:1] + c_ref[1, :, 0:1]
    mean = agg / jnp.maximum(cnt, 1.0)
    h = h_ref[...]
    pre = _MM(mean, wl_ref[...]) + bl_ref[...] + _MM(h, wr_ref[...])
    h2 = jnp.maximum(pre, 0.0) + h
    t = jnp.maximum(_MM(h2, ws1_ref[...]) + bs1_ref[...], 0.0)
    g = jnp.sum(t * ws2_ref[...], axis=1, keepdims=True) + bs2_ref[...]
    alpha = alpha_ref[...]
    out_ref[...] = alpha * rr_ref[...] + (1.0 - alpha) * g


_layer2_head = pl.pallas_call(
    _layer2_head_body,
    grid=(_GRID,),
    in_specs=[_agg_spec(), _agg_spec(), _rows_spec(H),
              _full_spec((D, H)), _full_spec((1, H)), _full_spec((D, H)),
              _rows_spec(1), _full_spec((H, H // 2)),
              _full_spec((1, H // 2)), _full_spec((1, H // 2)),
              _full_spec((1, 1)), _full_spec((1, 1))],
    out_specs=_rows_spec(1),
    out_shape=jax.ShapeDtypeStruct((N, 1), jnp.float32),
)


def kernel(x, edge_index, reranker_scores, Wp, bp, Wl0, bl0, Wr0, Wl1, bl1,
           Wr1, Ws1, bs1, Ws2, bs2, alpha_logit):
    zd = jnp.zeros((CHUNK, D), jnp.float32)
    ones = jnp.ones((CHUNK, D), jnp.float32)

    pad = EPAD - E
    psrc = jnp.arange(pad, dtype=jnp.int32) % N
    src1 = jnp.concatenate([edge_index[0], psrc]).reshape(-1, CHUNK)
    src1 = src1[_PERM].reshape(-1)
    dump = N + (jnp.arange(pad, dtype=jnp.int32) % (NP_ - N))
    dst2 = jnp.concatenate([edge_index[1], dump]).reshape(-1, CHUNK)
    dst2 = dst2[_PERM]

    cnt = _sc_cnt(dst2, zd, ones)
    xn, res = _norm_res(x, Wp, bp.reshape(1, H))
    agg0 = _sc_agg(xn, src1, dst2, zd)
    h1 = _sage_layer(agg0, cnt, xn, res, Wl0, bl0.reshape(1, H), Wr0)
    agg1 = _sc_agg(h1, src1, dst2, zd)
    alpha = jax.nn.sigmoid(alpha_logit).reshape(1, 1)
    out = _layer2_head(agg1, cnt, h1, Wl1, bl1.reshape(1, H), Wr1,
                       reranker_scores.reshape(N, 1), Ws1,
                       bs1.reshape(1, H // 2), Ws2.reshape(1, H // 2),
                       bs2.reshape(1, 1), alpha)
    return out[:, 0]


# speedup vs baseline: 1.1381x; 1.0185x over previous
import functools

import numpy as np

import jax
import jax.numpy as jnp
from jax import lax
from jax.experimental import pallas as pl
from jax.experimental.pallas import tpu as pltpu
from jax.experimental.pallas import tpu_sc as plsc

N = 10000
E = 320000
D = 128
H = 128

NC = 2
NS = 16
NW = NC * NS
CHUNK = 128
NCH_T = 80
EPAD = NW * NCH_T * CHUNK
NP_ = 10240
ROWS_PER_SUB = NP_ // NS

_MM = functools.partial(jnp.dot, precision=lax.Precision.HIGHEST,
                        preferred_element_type=jnp.float32)

_MESH = dict(core_axis_name="c", subcore_axis_name="s")

_NCHUNKS = EPAD // CHUNK
_PERM = np.arange(_NCHUNKS).reshape(NCH_T, NW).T.reshape(-1)


@functools.lru_cache(maxsize=None)
def _make_sc_agg():
    mesh = plsc.VectorSubcoreMesh(num_cores=NC, num_subcores=NS, **_MESH)
    scratch = [
        pltpu.VMEM((NCH_T, CHUNK), jnp.int32),
        pltpu.VMEM((CHUNK,), jnp.int32),
        pltpu.VMEM((CHUNK,), jnp.int32),
        pltpu.VMEM((CHUNK, D), jnp.float32),
        pltpu.VMEM((CHUNK, D), jnp.float32),
        pltpu.VMEM_SHARED((NP_, D), jnp.float32),
        pltpu.SemaphoreType.DMA,
        pltpu.SemaphoreType.DMA,
    ]

    @functools.partial(
        pl.kernel, mesh=mesh, scratch_types=scratch,
        out_type=jax.ShapeDtypeStruct((NC, NP_, D), jnp.float32))
    def sc_agg(h_hbm, src1_hbm, dst2_hbm, zd_hbm, agg_out,
               dst2v, src_a, src_b, rows_a, rows_b, acc_sh, sem_a, sem_b):
        cid = lax.axis_index("c")
        sid = lax.axis_index("s")
        wid = sid * NC + cid
        row0 = sid * ROWS_PER_SUB
        cbase = wid * NCH_T

        pltpu.sync_copy(zd_hbm, rows_a)

        @pl.loop(0, ROWS_PER_SUB // CHUNK)
        def _(i):
            pltpu.sync_copy(rows_a, acc_sh.at[pl.ds(row0 + i * CHUNK, CHUNK)])

        pltpu.sync_copy(dst2_hbm.at[pl.ds(cbase, NCH_T)], dst2v)
        plsc.subcore_barrier()

        def load_src(buf, c):
            pltpu.sync_copy(src1_hbm.at[pl.ds(c * CHUNK, CHUNK)], buf)

        def start_gather(idx_buf, rows_buf, sem):
            pltpu.async_copy(h_hbm.at[idx_buf], rows_buf, sem)

        def finish_gather(idx_buf, rows_buf, sem):
            pltpu.make_async_copy(h_hbm.at[idx_buf], rows_buf, sem).wait()

        def scatter(rows_buf, c):
            pltpu.sync_copy(rows_buf, acc_sh.at[dst2v.at[c - cbase]], add=True)

        load_src(src_a, cbase)
        start_gather(src_a, rows_a, sem_a)

        @pl.loop(0, NCH_T // 2 - 1)
        def _(p):
            c0 = cbase + 2 * p
            load_src(src_b, c0 + 1)
            start_gather(src_b, rows_b, sem_b)
            finish_gather(src_a, rows_a, sem_a)
            scatter(rows_a, c0)
            load_src(src_a, c0 + 2)
            start_gather(src_a, rows_a, sem_a)
            finish_gather(src_b, rows_b, sem_b)
            scatter(rows_b, c0 + 1)

        clast = cbase + NCH_T - 1
        load_src(src_b, clast)
        start_gather(src_b, rows_b, sem_b)
        finish_gather(src_a, rows_a, sem_a)
        scatter(rows_a, clast - 1)
        finish_gather(src_b, rows_b, sem_b)
        scatter(rows_b, clast)

        plsc.subcore_barrier()
        rows = pl.ds(row0, ROWS_PER_SUB)
        pltpu.sync_copy(acc_sh.at[rows], agg_out.at[cid, rows])

    return sc_agg


@functools.lru_cache(maxsize=None)
def _make_sc_cnt():
    mesh = plsc.VectorSubcoreMesh(num_cores=NC, num_subcores=NS, **_MESH)
    scratch = [
        pltpu.VMEM((NCH_T, CHUNK), jnp.int32),
        pltpu.VMEM((CHUNK, D), jnp.float32),
        pltpu.VMEM((CHUNK, D), jnp.float32),
        pltpu.VMEM_SHARED((NP_, D), jnp.float32),
        pltpu.SemaphoreType.DMA,
    ]

    @functools.partial(
        pl.kernel, mesh=mesh, scratch_types=scratch,
        out_type=jax.ShapeDtypeStruct((NC, NP_, D), jnp.float32))
    def sc_cnt(dst2_hbm, zd_hbm, ones_hbm, cnt_out,
               dst2v, ones_v, zbuf, cnt_sh, sem):
        cid = lax.axis_index("c")
        sid = lax.axis_index("s")
        wid = sid * NC + cid
        row0 = sid * ROWS_PER_SUB
        cbase = wid * NCH_T

        pltpu.sync_copy(zd_hbm, zbuf)
        pltpu.sync_copy(ones_hbm, ones_v)
        pltpu.sync_copy(dst2_hbm.at[pl.ds(cbase, NCH_T)], dst2v)

        @pl.loop(0, ROWS_PER_SUB // CHUNK)
        def _(i):
            pltpu.sync_copy(zbuf, cnt_sh.at[pl.ds(row0 + i * CHUNK, CHUNK)])

        plsc.subcore_barrier()

        @pl.loop(0, NCH_T // 16)
        def _(w):
            @pl.loop(0, 16)
            def _(i):
                pltpu.async_copy(ones_v, cnt_sh.at[dst2v.at[w * 16 + i]],
                                 sem, add=True)

            @pl.loop(0, 16)
            def _(i):
                pltpu.make_async_copy(ones_v, cnt_sh.at[dst2v.at[w * 16 + i]],
                                      sem).wait()

        plsc.subcore_barrier()
        rows = pl.ds(row0, ROWS_PER_SUB)
        pltpu.sync_copy(cnt_sh.at[rows], cnt_out.at[cid, rows])

    return sc_cnt


def _sc_agg(h, src2, dst2, zd):
    return _make_sc_agg()(h, src2, dst2, zd)


def _sc_cnt(dst2, zd, ones):
    return _make_sc_cnt()(dst2, zd, ones)


ROWS = 1000
_GRID = N // ROWS


def _rows_spec(minor):
    return pl.BlockSpec((ROWS, minor), lambda i: (i, 0))


def _full_spec(shape):
    nd = len(shape)
    return pl.BlockSpec(shape, lambda i, _nd=nd: (0,) * _nd)


def _norm_res_body(x_ref, wp_ref, bp_ref, xn_ref, res_ref):
    x = x_ref[...]
    nrm = jnp.sqrt(jnp.sum(x * x, axis=1, keepdims=True))
    xn = x / jnp.maximum(nrm, 1e-12)
    xn_ref[...] = xn
    res_ref[...] = _MM(xn, wp_ref[...]) + bp_ref[...]


_norm_res = pl.pallas_call(
    _norm_res_body,
    grid=(_GRID,),
    in_specs=[_rows_spec(D), _full_spec((D, H)), _full_spec((1, H))],
    out_specs=[_rows_spec(D), _rows_spec(H)],
    out_shape=[jax.ShapeDtypeStruct((N, D), jnp.float32),
               jax.ShapeDtypeStruct((N, H), jnp.float32)],
)


def _layer_body(a_ref, c_ref, h_ref, res_ref, wl_ref, bl_ref, wr_ref, out_ref):
    agg = a_ref[0] + a_ref[1]
    cnt = c_ref[0, :, 0:1] + c_ref[1, :, 0:1]
    mean = agg / jnp.maximum(cnt, 1.0)
    h = h_ref[...]
    pre = _MM(mean, wl_ref[...]) + bl_ref[...] + _MM(h, wr_ref[...])
    out_ref[...] = jnp.maximum(pre, 0.0) + res_ref[...]


def _agg_spec():
    return pl.BlockSpec((NC, ROWS, D), lambda i: (0, i, 0))


_sage_layer = pl.pallas_call(
    _layer_body,
    grid=(_GRID,),
    in_specs=[_agg_spec(), _agg_spec(), _rows_spec(D), _rows_spec(H),
              _full_spec((D, H)), _full_spec((1, H)), _full_spec((D, H))],
    out_specs=_rows_spec(H),
    out_shape=jax.ShapeDtypeStruct((N, H), jnp.float32),
)


def _head_body(h2_ref, rr_ref, ws1_ref, bs1_ref, ws2_ref, bs2_ref,
               alpha_ref, out_ref):
    h2 = h2_ref[...]
    t = jnp.maximum(_MM(h2, ws1_ref[...]) + bs1_ref[...], 0.0)
    g = jnp.sum(t * ws2_ref[...], axis=1, keepdims=True) + bs2_ref[...]
    alpha = alpha_ref[...]
    out_ref[...] = alpha * rr_ref[...] + (1.0 - alpha) * g


_head = pl.pallas_call(
    _head_body,
    grid=(_GRID,),
    in_specs=[_rows_spec(H), _rows_spec(1), _full_spec((H, H // 2)),
              _full_spec((1, H // 2)), _full_spec((1, H // 2)),
              _full_spec((1, 1)), _full_spec((1, 1))],
    out_specs=_rows_spec(1),
    out_shape=jax.ShapeDtypeStruct((N, 1), jnp.float32),
)


def kernel(x, edge_index, reranker_scores, Wp, bp, Wl0, bl0, Wr0, Wl1, bl1,
           Wr1, Ws1, bs1, Ws2, bs2, alpha_logit):
    zd = jnp.zeros((CHUNK, D), jnp.float32)
    ones = jnp.ones((CHUNK, D), jnp.float32)

    pad = EPAD - E
    psrc = jnp.arange(pad, dtype=jnp.int32) % N
    src1 = jnp.concatenate([edge_index[0], psrc]).reshape(-1, CHUNK)
    src1 = src1[_PERM].reshape(-1)
    dump = N + (jnp.arange(pad, dtype=jnp.int32) % (NP_ - N))
    dst2 = jnp.concatenate([edge_index[1], dump]).reshape(-1, CHUNK)
    dst2 = dst2[_PERM]

    cnt = _sc_cnt(dst2, zd, ones)
    xn, res = _norm_res(x, Wp, bp.reshape(1, H))
    agg0 = _sc_agg(xn, src1, dst2, zd)
    h1 = _sage_layer(agg0, cnt, xn, res, Wl0, bl0.reshape(1, H), Wr0)
    agg1 = _sc_agg(h1, src1, dst2, zd)
    h2 = _sage_layer(agg1, cnt, h1, h1, Wl1, bl1.reshape(1, H), Wr1)

    alpha = jax.nn.sigmoid(alpha_logit).reshape(1, 1)
    out = _head(h2, reranker_scores.reshape(N, 1), Ws1, bs1.reshape(1, H // 2),
                Ws2.reshape(1, H // 2), bs2.reshape(1, 1), alpha)
    return out[:, 0]
